# baseline (speedup 1.0000x reference)
import jax
import jax.numpy as jnp
from jax.experimental import pallas as pl
from jax.experimental.pallas import tpu as pltpu

_NUM_POINTS = 64
_FOLD = 8


def _fps_body(far_ref, pnc_ref, pt4_ref, sampled_ref, cent_ref):
    N = pnc_ref.shape[1]
    C = pnc_ref.shape[2]
    NL = N // _FOLD

    n_iota = (jax.lax.broadcasted_iota(jnp.int32, (_FOLD, NL), 0) * NL
              + jax.lax.broadcasted_iota(jnp.int32, (_FOLD, NL), 1))
    eye_mask = (jax.lax.broadcasted_iota(jnp.int32, (C, C), 0)
                == jax.lax.broadcasted_iota(jnp.int32, (C, C), 1))
    lane_np = jax.lax.broadcasted_iota(jnp.int32, (1, _NUM_POINTS), 1)

    pt4 = pt4_ref[0]

    def body(i, carry):
        f, distance, cent_vec = carry
        cent_vec = jnp.where(lane_np == i, f, cent_vec)
        row = pnc_ref[0, pl.ds(f, 1), :]
        sampled_ref[0, pl.ds(i, 1), :] = row
        col = jnp.sum(
            jnp.where(eye_mask, jnp.broadcast_to(row, (C, C)), 0.0),
            axis=1, keepdims=True)
        diff = pt4 - col
        dist = jnp.sum(diff * diff, axis=1)
        distance = jnp.where(dist < distance, dist, distance)
        m = jnp.max(distance)
        f_new = jnp.min(jnp.where(distance == m, n_iota, jnp.int32(N)))
        return f_new, distance, cent_vec

    f0 = far_ref[pl.program_id(0)]
    dist0 = jnp.full((_FOLD, NL), 1e10, jnp.float32)
    cent0 = jnp.zeros((1, _NUM_POINTS), jnp.int32)
    _, _, cent_vec = jax.lax.fori_loop(0, _NUM_POINTS, body, (f0, dist0, cent0))
    cent_ref[0] = cent_vec


def _fps_pallas(points, far0, interpret=False):
    B, N, C = points.shape
    NL = N // _FOLD
    pt4 = points.transpose(0, 2, 1).reshape(B, C, _FOLD, NL).transpose(0, 2, 1, 3)
    sampled, cent = pl.pallas_call(
        _fps_body,
        grid=(B,),
        in_specs=[
            pl.BlockSpec(memory_space=pltpu.SMEM),
            pl.BlockSpec((1, N, C), lambda b: (b, 0, 0)),
            pl.BlockSpec((1, _FOLD, C, NL), lambda b: (b, 0, 0, 0)),
        ],
        out_specs=[
            pl.BlockSpec((1, _NUM_POINTS, C), lambda b: (b, 0, 0)),
            pl.BlockSpec((1, 1, _NUM_POINTS), lambda b: (b, 0, 0)),
        ],
        out_shape=[
            jax.ShapeDtypeStruct((B, _NUM_POINTS, C), jnp.float32),
            jax.ShapeDtypeStruct((B, 1, _NUM_POINTS), jnp.int32),
        ],
        compiler_params=pltpu.CompilerParams(
            dimension_semantics=("parallel",)),
        interpret=interpret,
    )(far0, points, pt4)
    return sampled, cent.reshape(B, _NUM_POINTS)


@jax.jit
def kernel(features):
    B = features.shape[0]
    C = features.shape[-1]
    points = features.reshape(B, -1, C)
    N = points.shape[1]
    far0 = jax.random.randint(jax.random.key(1), (B,), 0, N, dtype=jnp.int32)
    return _fps_pallas(points, far0)


# speedup vs baseline: 1.2475x; 1.2475x over previous
import jax
import jax.numpy as jnp
from jax.experimental import pallas as pl
from jax.experimental.pallas import tpu as pltpu

_NUM_POINTS = 64
_FOLD = 8


def _fps_body(far_ref, pnc_ref, pt4_ref, sampled_ref, cent_ref):
    N = pnc_ref.shape[1]
    C = pnc_ref.shape[2]
    NL = N // _FOLD

    n_iota = (jax.lax.broadcasted_iota(jnp.int32, (_FOLD, NL), 0) * NL
              + jax.lax.broadcasted_iota(jnp.int32, (_FOLD, NL), 1))
    eye_mask = (jax.lax.broadcasted_iota(jnp.int32, (C, C), 0)
                == jax.lax.broadcasted_iota(jnp.int32, (C, C), 1))
    lane_np = jax.lax.broadcasted_iota(jnp.int32, (1, _NUM_POINTS), 1)

    chunk_l = 256

    def body(i, carry):
        f, distance, cent_vec = carry
        cent_vec = jnp.where(lane_np == i, f, cent_vec)
        row = pnc_ref[0, pl.ds(f, 1), :]
        sampled_ref[0, pl.ds(i, 1), :] = row
        col = jnp.sum(
            jnp.where(eye_mask, jnp.broadcast_to(row, (C, C)), 0.0),
            axis=1, keepdims=True)
        parts = []
        for lb in range(0, NL, chunk_l):
            chunk = pt4_ref[0, :, :, lb:lb + chunk_l]
            d = chunk - col
            parts.append(jnp.sum(d * d, axis=1))
        dist = jnp.concatenate(parts, axis=1)
        distance = jnp.where(dist < distance, dist, distance)
        m = jnp.max(distance)
        f_new = jnp.min(jnp.where(distance == m, n_iota, jnp.int32(N)))
        return f_new, distance, cent_vec

    f0 = far_ref[pl.program_id(0)]
    dist0 = jnp.full((_FOLD, NL), 1e10, jnp.float32)
    cent0 = jnp.zeros((1, _NUM_POINTS), jnp.int32)
    _, _, cent_vec = jax.lax.fori_loop(0, _NUM_POINTS, body, (f0, dist0, cent0))
    cent_ref[0] = cent_vec


def _fps_pallas(points, far0, interpret=False):
    B, N, C = points.shape
    NL = N // _FOLD
    pt4 = points.transpose(0, 2, 1).reshape(B, C, _FOLD, NL).transpose(0, 2, 1, 3)
    sampled, cent = pl.pallas_call(
        _fps_body,
        grid=(B,),
        in_specs=[
            pl.BlockSpec(memory_space=pltpu.SMEM),
            pl.BlockSpec((1, N, C), lambda b: (b, 0, 0)),
            pl.BlockSpec((1, _FOLD, C, NL), lambda b: (b, 0, 0, 0)),
        ],
        out_specs=[
            pl.BlockSpec((1, _NUM_POINTS, C), lambda b: (b, 0, 0)),
            pl.BlockSpec((1, 1, _NUM_POINTS), lambda b: (b, 0, 0)),
        ],
        out_shape=[
            jax.ShapeDtypeStruct((B, _NUM_POINTS, C), jnp.float32),
            jax.ShapeDtypeStruct((B, 1, _NUM_POINTS), jnp.int32),
        ],
        compiler_params=pltpu.CompilerParams(
            dimension_semantics=("parallel",)),
        interpret=interpret,
    )(far0, points, pt4)
    return sampled, cent.reshape(B, _NUM_POINTS)


@jax.jit
def kernel(features):
    B = features.shape[0]
    C = features.shape[-1]
    points = features.reshape(B, -1, C)
    N = points.shape[1]
    far0 = jax.random.randint(jax.random.key(1), (B,), 0, N, dtype=jnp.int32)
    return _fps_pallas(points, far0)


# speedup vs baseline: 1.4360x; 1.1511x over previous
import jax
import jax.numpy as jnp
from jax.experimental import pallas as pl
from jax.experimental.pallas import tpu as pltpu

_NUM_POINTS = 64
_FOLD = 8


def _fps_body(far_ref, pnc_ref, pt4_ref, sampled_ref, cent_ref):
    NB = pnc_ref.shape[0]
    N = pnc_ref.shape[1]
    C = pnc_ref.shape[2]
    NL = N // _FOLD

    n_iota = (jax.lax.broadcasted_iota(jnp.int32, (_FOLD, NL), 0) * NL
              + jax.lax.broadcasted_iota(jnp.int32, (_FOLD, NL), 1))
    eye_mask = (jax.lax.broadcasted_iota(jnp.int32, (C, C), 0)
                == jax.lax.broadcasted_iota(jnp.int32, (C, C), 1))
    lane_np = jax.lax.broadcasted_iota(jnp.int32, (1, _NUM_POINTS), 1)

    chunk_l = 256

    def gather_row(bb, i, f):
        row = pnc_ref[bb, pl.ds(f, 1), :]
        sampled_ref[bb, pl.ds(i, 1), :] = row
        return jnp.sum(
            jnp.where(eye_mask, jnp.broadcast_to(row, (C, C)), 0.0),
            axis=1, keepdims=True)

    def body(i, carry):
        fs, dists, cents = carry
        new_f, new_d, new_c = [], [], []
        cols = []
        for bb in range(NB):
            new_c.append(jnp.where(lane_np == i, fs[bb], cents[bb]))
            cols.append(gather_row(bb, i, fs[bb]))
        for bb in range(NB):
            parts = []
            for lb in range(0, NL, chunk_l):
                chunk = pt4_ref[bb, :, :, lb:lb + chunk_l]
                d = chunk - cols[bb]
                parts.append(jnp.sum(d * d, axis=1))
            dist = jnp.concatenate(parts, axis=1)
            distance = jnp.where(dist < dists[bb], dist, dists[bb])
            m = jnp.max(distance)
            new_f.append(jnp.min(
                jnp.where(distance == m, n_iota, jnp.int32(N))))
            new_d.append(distance)
        return tuple(new_f), tuple(new_d), tuple(new_c)

    pid = pl.program_id(0)
    fs0 = tuple(far_ref[pid * NB + bb] for bb in range(NB))
    dist0 = tuple(jnp.full((_FOLD, NL), 1e10, jnp.float32) for _ in range(NB))
    cent0 = tuple(jnp.zeros((1, _NUM_POINTS), jnp.int32) for _ in range(NB))
    fs, _, cents = jax.lax.fori_loop(
        0, _NUM_POINTS - 1, body, (fs0, dist0, cent0))
    for bb in range(NB):
        cent_ref[bb] = jnp.where(lane_np == (_NUM_POINTS - 1), fs[bb],
                                 cents[bb])
        gather_row(bb, _NUM_POINTS - 1, fs[bb])


_NB = 2


def _fps_pallas(points, far0, interpret=False):
    B, N, C = points.shape
    NL = N // _FOLD
    pt4 = points.transpose(0, 2, 1).reshape(B, C, _FOLD, NL).transpose(0, 2, 1, 3)
    sampled, cent = pl.pallas_call(
        _fps_body,
        grid=(B // _NB,),
        in_specs=[
            pl.BlockSpec(memory_space=pltpu.SMEM),
            pl.BlockSpec((_NB, N, C), lambda b: (b, 0, 0)),
            pl.BlockSpec((_NB, _FOLD, C, NL), lambda b: (b, 0, 0, 0)),
        ],
        out_specs=[
            pl.BlockSpec((_NB, _NUM_POINTS, C), lambda b: (b, 0, 0)),
            pl.BlockSpec((_NB, 1, _NUM_POINTS), lambda b: (b, 0, 0)),
        ],
        out_shape=[
            jax.ShapeDtypeStruct((B, _NUM_POINTS, C), jnp.float32),
            jax.ShapeDtypeStruct((B, 1, _NUM_POINTS), jnp.int32),
        ],
        compiler_params=pltpu.CompilerParams(
            dimension_semantics=("parallel",)),
        interpret=interpret,
    )(far0, points, pt4)
    return sampled, cent.reshape(B, _NUM_POINTS)


@jax.jit
def kernel(features):
    B = features.shape[0]
    C = features.shape[-1]
    points = features.reshape(B, -1, C)
    N = points.shape[1]
    far0 = jax.random.randint(jax.random.key(1), (B,), 0, N, dtype=jnp.int32)
    return _fps_pallas(points, far0)


# speedup vs baseline: 1.9028x; 1.3251x over previous
import jax
import jax.numpy as jnp
from jax.experimental import pallas as pl
from jax.experimental.pallas import tpu as pltpu

_NUM_POINTS = 64
_FOLD = 8


def _fps_body(far_ref, pt7_any, pnc_any, sampled_ref, cent_ref,
              pt7_ref, rows_smem, rows_vmem, copy_sem, smem_sems, vmem_sems):
    copy = pltpu.make_async_copy(pt7_any, pt7_ref, copy_sem)
    copy.start()
    NB = pt7_ref.shape[0]
    C = pt7_ref.shape[1]
    NL = pt7_ref.shape[3]
    N = _FOLD * NL

    n_iota = (jax.lax.broadcasted_iota(jnp.int32, (_FOLD, NL), 0) * NL
              + jax.lax.broadcasted_iota(jnp.int32, (_FOLD, NL), 1))
    lane_np = jax.lax.broadcasted_iota(jnp.int32, (1, _NUM_POINTS), 1)

    chunk_l = 512

    def start_row_dma(bb, f):
        pltpu.make_async_copy(
            pnc_any.at[bb, pl.ds(f, 1), :],
            rows_smem.at[bb], smem_sems.at[bb]).start()
        pltpu.make_async_copy(
            pnc_any.at[bb, pl.ds(f, 1), :],
            rows_vmem.at[bb], vmem_sems.at[bb]).start()

    def wait_row_dma(bb, f):
        pltpu.make_async_copy(pnc_any.at[bb, pl.ds(f, 1), :],
                              rows_smem.at[bb], smem_sems.at[bb]).wait()
        pltpu.make_async_copy(pnc_any.at[bb, pl.ds(f, 1), :],
                              rows_vmem.at[bb], vmem_sems.at[bb]).wait()

    def body(i, carry):
        fs, dists, cents = carry
        new_f, new_d, new_c = [], [], []
        for bb in range(NB):
            wait_row_dma(bb, fs[bb])
        for bb in range(NB):
            new_c.append(jnp.where(lane_np == i, fs[bb], cents[bb]))
            sampled_ref[bb, pl.ds(i, 1), :] = rows_vmem[bb]
            parts = []
            for lb in range(0, NL, chunk_l):
                accs = [None] * 8
                for cg in range(C // 8):
                    for cs in range(8):
                        c = cg * 8 + cs
                        slab = pt7_ref[bb, c, :, lb:lb + chunk_l]
                        d = slab - rows_smem[bb, 0, c]
                        sq = d * d
                        accs[cs] = sq if cg == 0 else accs[cs] + sq
                parts.append(((accs[0] + accs[4]) + (accs[2] + accs[6]))
                             + ((accs[1] + accs[5]) + (accs[3] + accs[7])))
            dist = jnp.concatenate(parts, axis=1)
            distance = jnp.where(dist < dists[bb], dist, dists[bb])
            m = jnp.max(distance)
            f_new = jnp.min(jnp.where(distance == m, n_iota, jnp.int32(N)))
            new_f.append(f_new)
            new_d.append(distance)
        for bb in range(NB):
            start_row_dma(bb, new_f[bb])
        return tuple(new_f), tuple(new_d), tuple(new_c)

    copy.wait()
    fs0 = tuple(far_ref[bb] for bb in range(NB))
    for bb in range(NB):
        start_row_dma(bb, fs0[bb])
    dist0 = tuple(jnp.full((_FOLD, NL), 1e10, jnp.float32) for _ in range(NB))
    cent0 = tuple(jnp.zeros((1, _NUM_POINTS), jnp.int32) for _ in range(NB))
    fs, _, cents = jax.lax.fori_loop(
        0, _NUM_POINTS - 1, body, (fs0, dist0, cent0))
    last = _NUM_POINTS - 1
    for bb in range(NB):
        cent_ref[bb] = jnp.where(lane_np == last, fs[bb], cents[bb])
        wait_row_dma(bb, fs[bb])
        sampled_ref[bb, pl.ds(last, 1), :] = rows_vmem[bb]


def _fps_pallas(points, far0, interpret=False):
    B, N, C = points.shape
    NL = N // _FOLD
    pt7 = points.transpose(0, 2, 1).reshape(B, C, _FOLD, NL)
    sampled, cent = pl.pallas_call(
        _fps_body,
        grid=(1,),
        in_specs=[
            pl.BlockSpec(memory_space=pltpu.SMEM),
            pl.BlockSpec(memory_space=pl.ANY),
            pl.BlockSpec(memory_space=pl.ANY),
        ],
        out_specs=[
            pl.BlockSpec((B, _NUM_POINTS, C), lambda b: (0, 0, 0)),
            pl.BlockSpec((B, 1, _NUM_POINTS), lambda b: (0, 0, 0)),
        ],
        out_shape=[
            jax.ShapeDtypeStruct((B, _NUM_POINTS, C), jnp.float32),
            jax.ShapeDtypeStruct((B, 1, _NUM_POINTS), jnp.int32),
        ],
        scratch_shapes=[
            pltpu.VMEM((B, C, _FOLD, NL), jnp.float32),
            pltpu.SMEM((B, 1, C), jnp.float32),
            pltpu.VMEM((B, 1, C), jnp.float32),
            pltpu.SemaphoreType.DMA,
            pltpu.SemaphoreType.DMA((B,)),
            pltpu.SemaphoreType.DMA((B,)),
        ],
        compiler_params=pltpu.CompilerParams(
            dimension_semantics=("arbitrary",)),
        interpret=interpret,
    )(far0, pt7, points)
    return sampled, cent.reshape(B, _NUM_POINTS)


@jax.jit
def kernel(features):
    B = features.shape[0]
    C = features.shape[-1]
    points = features.reshape(B, -1, C)
    N = points.shape[1]
    far0 = jax.random.randint(jax.random.key(1), (B,), 0, N, dtype=jnp.int32)
    return _fps_pallas(points, far0)
